# Initial kernel scaffold; baseline (speedup 1.0000x reference)
import jax
import jax.numpy as jnp
from jax.experimental import pallas as pl


def kernel(h, edge_index, W1, b1, W2, b2):
    raise NotImplementedError("write your pallas kernel here")


# speedup vs baseline: 2.3187x; 2.3187x over previous
import functools

import jax
import jax.numpy as jnp
from jax import lax
from jax.experimental import pallas as pl
from jax.experimental.pallas import tpu as pltpu
from jax.experimental.pallas import tpu_sc as plsc

N = 10000
E = 320000
D = 128
DH = D // 2

K = 128
TILES = 16
CHUNKS = 2560
EP = CHUNKS * K
CT = CHUNKS // TILES

NP_ = 10240
RPT = NP_ // TILES

BR = 256
GRID = (N + BR - 1) // BR

_mesh = plsc.VectorSubcoreMesh(core_axis_name="c", subcore_axis_name="s")


def _deg_body(stacked, out, idx_v, ones_v, buf_v, deg_sp, sem):
    c = lax.axis_index("c")
    s = lax.axis_index("s")

    pltpu.sync_copy(stacked.at[c, pl.ds(s * CT, CT)], idx_v)

    ones16 = jnp.ones((16,), jnp.float32)
    for v in range(K // 16):
        ones_v[pl.ds(v * 16, 16)] = ones16
    zero16 = jnp.zeros((16,), jnp.float32)
    for v in range(RPT // 16):
        buf_v[pl.ds(v * 16, 16)] = zero16

    pltpu.sync_copy(buf_v, deg_sp.at[pl.ds(s * RPT, RPT)])
    plsc.subcore_barrier()

    def group(g, carry):
        for k in range(8):
            pltpu.async_copy(ones_v, deg_sp.at[idx_v.at[g * 8 + k]], sem,
                             add=True)
        for k in range(8):
            pltpu.make_async_copy(ones_v, deg_sp.at[pl.ds(0, K)], sem).wait()
        return carry

    lax.fori_loop(0, CT // 8, group, 0)

    plsc.subcore_barrier()
    pltpu.sync_copy(deg_sp.at[pl.ds(s * RPT, RPT)], buf_v)
    pltpu.sync_copy(buf_v, out.at[c, pl.ds(s * RPT, RPT)])


_deg = pl.kernel(
    _deg_body,
    out_type=jax.ShapeDtypeStruct((2, NP_), jnp.float32),
    mesh=_mesh,
    scratch_types=[
        pltpu.VMEM((CT, K), jnp.int32),
        pltpu.VMEM((K,), jnp.float32),
        pltpu.VMEM((RPT,), jnp.float32),
        pltpu.VMEM_SHARED((NP_,), jnp.float32),
        pltpu.SemaphoreType.DMA,
    ],
)


NH = 5120
NR = 5248
RT = NR // TILES


def _scat_body(y, srcp, dstp, out, idx_v, dst_v, rows_a, rows_b,
               sem_a, sem_b, agg_sp):
    c = lax.axis_index("c")
    s = lax.axis_index("s")

    pltpu.sync_copy(srcp.at[pl.ds(s * CT, CT)], idx_v)
    pltpu.sync_copy(dstp.at[pl.ds(s * CT, CT)], dst_v)

    base = c * NH

    def tconv(j, carry):
        for v in range(K // 16):
            sl = pl.ds(v * 16, 16)
            loc = dst_v[j, sl] - base
            ok = (loc >= 0) & (loc < NH)
            dst_v[j, sl] = jnp.where(ok, loc, NH)
        return carry

    lax.fori_loop(0, CT, tconv, 0)

    zero16 = jnp.zeros((16,), jnp.float32)

    def zrow(r, carry):
        for v in range(D // 16):
            rows_a[r, pl.ds(v * 16, 16)] = zero16
        return carry

    lax.fori_loop(0, K, zrow, 0)
    pltpu.sync_copy(rows_a, agg_sp.at[pl.ds(s * RT, K)])
    pltpu.sync_copy(rows_a, agg_sp.at[pl.ds(s * RT + K, K)])
    pltpu.sync_copy(rows_a.at[pl.ds(0, RT - 2 * K)],
                    agg_sp.at[pl.ds(s * RT + 2 * K, RT - 2 * K)])
    plsc.subcore_barrier()

    pltpu.async_copy(y.at[idx_v.at[0]], rows_a, sem_a)

    def step(g, carry):
        i0 = 2 * g
        pltpu.async_copy(y.at[idx_v.at[i0 + 1]], rows_b, sem_b)
        pltpu.make_async_copy(y.at[pl.ds(0, K)], rows_a, sem_a).wait()
        pltpu.sync_copy(rows_a, agg_sp.at[dst_v.at[i0]], add=True)
        pltpu.async_copy(y.at[idx_v.at[i0 + 2]], rows_a, sem_a)
        pltpu.make_async_copy(y.at[pl.ds(0, K)], rows_b, sem_b).wait()
        pltpu.sync_copy(rows_b, agg_sp.at[dst_v.at[i0 + 1]], add=True)
        return carry

    lax.fori_loop(0, (CT - 2) // 2, step, 0)
    pltpu.async_copy(y.at[idx_v.at[CT - 1]], rows_b, sem_b)
    pltpu.make_async_copy(y.at[pl.ds(0, K)], rows_a, sem_a).wait()
    pltpu.sync_copy(rows_a, agg_sp.at[dst_v.at[CT - 2]], add=True)
    pltpu.make_async_copy(y.at[pl.ds(0, K)], rows_b, sem_b).wait()
    pltpu.sync_copy(rows_b, agg_sp.at[dst_v.at[CT - 1]], add=True)

    plsc.subcore_barrier()
    for k in range(2):
        pltpu.sync_copy(agg_sp.at[pl.ds(s * RT + k * K, K)], rows_a)
        pltpu.sync_copy(rows_a, out.at[c, pl.ds(s * RT + k * K, K)])
    pltpu.sync_copy(agg_sp.at[pl.ds(s * RT + 2 * K, RT - 2 * K)],
                    rows_a.at[pl.ds(0, RT - 2 * K)])
    pltpu.sync_copy(rows_a.at[pl.ds(0, RT - 2 * K)],
                    out.at[c, pl.ds(s * RT + 2 * K, RT - 2 * K)])


_scat = pl.kernel(
    _scat_body,
    out_type=jax.ShapeDtypeStruct((2, NR, D), jnp.float32),
    mesh=_mesh,
    scratch_types=[
        pltpu.VMEM((CT, K), jnp.int32),
        pltpu.VMEM((CT, K), jnp.int32),
        pltpu.VMEM((K, D), jnp.float32),
        pltpu.VMEM((K, D), jnp.float32),
        pltpu.SemaphoreType.DMA,
        pltpu.SemaphoreType.DMA,
        pltpu.VMEM_SHARED((NR, D), jnp.float32),
    ],
)


def _inv_sqrt_deg(d):
    return jnp.where(d > 0, lax.rsqrt(jnp.maximum(d, 1.0)), 0.0)


def _mm1_body(x_ref, dout_ref, w_ref, o_ref):
    no = _inv_sqrt_deg(dout_ref[...])
    o_ref[...] = jnp.dot(x_ref[...] * no, w_ref[...],
                         preferred_element_type=jnp.float32,
                         precision=lax.Precision.HIGHEST)


def _mid_body(agg_ref, din_ref, dout_ref, b_ref, w_ref, o_ref):
    a = agg_ref[0]
    ni = _inv_sqrt_deg(din_ref[...])
    h1 = jnp.maximum(a * ni + b_ref[...], 0.0)
    no = _inv_sqrt_deg(dout_ref[...])
    o_ref[...] = jnp.dot(h1 * no, w_ref[...],
                         preferred_element_type=jnp.float32,
                         precision=lax.Precision.HIGHEST)


def _fin_body(agg_ref, din_ref, b_ref, o_ref):
    a = agg_ref[0]
    ni = _inv_sqrt_deg(din_ref[...])
    o_ref[...] = jnp.maximum(a * ni + b_ref[...], 0.0)


_col_spec = pl.BlockSpec((BR, 1), lambda i: (i, 0))
_row_spec = pl.BlockSpec((BR, D), lambda i: (i, 0))
_agg_spec = pl.BlockSpec((1, BR, D),
                         lambda i: (i // (NH // BR), i % (NH // BR), 0))
_w_spec = pl.BlockSpec((D, D), lambda i: (0, 0))
_b_spec = pl.BlockSpec((1, D), lambda i: (0, 0))
_out_sds = jax.ShapeDtypeStruct((N, D), jnp.float32)

_mm1 = pl.pallas_call(
    _mm1_body, grid=(GRID,),
    in_specs=[_row_spec, _col_spec, _w_spec],
    out_specs=_row_spec, out_shape=_out_sds)

_mid = pl.pallas_call(
    _mid_body, grid=(GRID,),
    in_specs=[_agg_spec, _col_spec, _col_spec, _b_spec, _w_spec],
    out_specs=_row_spec, out_shape=_out_sds)

_fin = pl.pallas_call(
    _fin_body, grid=(GRID,),
    in_specs=[_agg_spec, _col_spec, _b_spec],
    out_specs=_row_spec, out_shape=_out_sds)


def kernel(h, edge_index, W1, b1, W2, b2):
    src = edge_index[0].astype(jnp.int32)
    dst = edge_index[1].astype(jnp.int32)
    npad = EP - E
    trash = jnp.full((npad,), N, jnp.int32)
    src_scat = jnp.concatenate([src, jnp.zeros((npad,), jnp.int32)])
    src_scat = src_scat.reshape(CHUNKS, K)
    dst_pad = jnp.concatenate([dst, trash]).reshape(CHUNKS, K)
    src_deg = jnp.concatenate([src, trash]).reshape(CHUNKS, K)

    degs = _deg(jnp.stack([src_deg, dst_pad]))
    dout = degs[0].reshape(NP_, 1)
    din = degs[1].reshape(NP_, 1)

    b1r = b1.reshape(1, D)
    b2r = b2.reshape(1, D)

    y1 = _mm1(h, dout, W1)
    agg1 = _scat(y1, src_scat, dst_pad)
    y2 = _mid(agg1, din, dout, b1r, W2)
    agg2 = _scat(y2, src_scat, dst_pad)
    return _fin(agg2, din, b2r)


# speedup vs baseline: 3.3388x; 1.4400x over previous
import functools

import jax
import jax.numpy as jnp
from jax import lax
from jax.experimental import pallas as pl
from jax.experimental.pallas import tpu as pltpu
from jax.experimental.pallas import tpu_sc as plsc

N = 10000
E = 320000
D = 128
DH = D // 2

K = 128
TILES = 16
CHUNKS = 2560
EP = CHUNKS * K
CT = CHUNKS // TILES

NP_ = 10240
RPT = NP_ // TILES

BR = 256
GRID = (N + BR - 1) // BR

_mesh = plsc.VectorSubcoreMesh(core_axis_name="c", subcore_axis_name="s")


NDB = 8


def _deg_body(pkp, out, pk_v, ones_v, buf_v, i0, i1, i2, i3, i4, i5, i6, i7,
              deg_sp, sem):
    idxb = (i0, i1, i2, i3, i4, i5, i6, i7)
    c = lax.axis_index("c")
    s = lax.axis_index("s")

    pltpu.sync_copy(pkp.at[pl.ds(s * CT * K, CT * K)], pk_v)

    ones16 = jnp.ones((16,), jnp.float32)
    for v in range(K // 16):
        ones_v[pl.ds(v * 16, 16)] = ones16
    zero16 = jnp.zeros((16,), jnp.float32)
    for v in range(RPT // 16):
        buf_v[pl.ds(v * 16, 16)] = zero16

    pltpu.sync_copy(buf_v, deg_sp.at[pl.ds(s * RPT, RPT)])
    plsc.subcore_barrier()

    def unpack(i, k):
        for v in range(K // 16):
            p = pk_v[pl.ds(i * K + v * 16, 16)]
            d = p & 16383
            pad = d >= N
            trash = N + s * 8 + v
            val = jnp.where(c == 0, jnp.right_shift(p, 14), d)
            idxb[k][pl.ds(v * 16, 16)] = jnp.where(pad, trash, val)

    def group(g, carry):
        for k in range(NDB):
            unpack(g * NDB + k, k)
            pltpu.async_copy(ones_v, deg_sp.at[idxb[k]], sem, add=True)
        for k in range(NDB):
            pltpu.make_async_copy(ones_v, deg_sp.at[pl.ds(0, K)], sem).wait()
        return carry

    lax.fori_loop(0, CT // NDB, group, 0)

    plsc.subcore_barrier()
    pltpu.sync_copy(deg_sp.at[pl.ds(s * RPT, RPT)], buf_v)
    pltpu.sync_copy(buf_v, out.at[c, pl.ds(s * RPT, RPT)])


_deg = pl.kernel(
    _deg_body,
    out_type=jax.ShapeDtypeStruct((2, NP_), jnp.float32),
    mesh=_mesh,
    scratch_types=(
        [pltpu.VMEM((CT * K,), jnp.int32),
         pltpu.VMEM((K,), jnp.float32),
         pltpu.VMEM((RPT,), jnp.float32)]
        + [pltpu.VMEM((K,), jnp.int32) for _ in range(NDB)]
        + [pltpu.VMEM_SHARED((NP_,), jnp.float32),
           pltpu.SemaphoreType.DMA]
    ),
)


NH = 5120
NR = 5248
RT = NR // TILES


NB = 4


SLAB = CT * K


def _scat_body(y, pkp, out, pk_v,
               gi0, gi1, gi2, gi3, si0, si1, si2, si3,
               r0, r1, r2, r3, g0, g1, g2, g3, s0, s1, s2, s3, agg_sp):
    gis = (gi0, gi1, gi2, gi3)
    sis = (si0, si1, si2, si3)
    rows = (r0, r1, r2, r3)
    gs = (g0, g1, g2, g3)
    ss = (s0, s1, s2, s3)
    c = lax.axis_index("c")
    s = lax.axis_index("s")

    pltpu.sync_copy(pkp.at[pl.ds(s * SLAB, SLAB)], pk_v.at[pl.ds(0, SLAB)])
    base = c * NH

    padvec = jnp.zeros((16,), jnp.int32) + (s * 16384 + N)
    iota16 = lax.iota(jnp.int32, 16)

    def cbody(j, cnt):
        for v in range(K // 16):
            p = pk_v[pl.ds(j * K + v * 16, 16)]
            loc = (p & 16383) - base
            m = (loc >= 0) & (loc < NH)
            mi = m.astype(jnp.int32)
            pos = cnt + plsc.cumsum(mi) - mi
            plsc.store_scatter(pk_v, [pos], p, mask=m)
            cnt = cnt + plsc.all_reduce_population_count(m)[0]
        return cnt

    cnt = lax.fori_loop(0, CT, cbody, 0)
    for t in range(NB * K // 16):
        plsc.store_scatter(pk_v, [cnt + 16 * t + iota16], padvec)
    nch = ((cnt + NB * K) // (NB * K)) * NB

    def unpack(i, b):
        for v in range(K // 16):
            sl = pl.ds(i * K + v * 16, 16)
            p = pk_v[sl]
            osl = pl.ds(v * 16, 16)
            gis[b][osl] = jnp.right_shift(p, 14)
            loc = (p & 16383) - base
            ok = (loc >= 0) & (loc < NH)
            sis[b][osl] = jnp.where(ok, loc, NH + s * 8 + v)

    zero16 = jnp.zeros((16,), jnp.float32)

    def zrow(r, carry):
        for v in range(D // 16):
            r0[r, pl.ds(v * 16, 16)] = zero16
        return carry

    lax.fori_loop(0, K, zrow, 0)
    pltpu.sync_copy(r0, agg_sp.at[pl.ds(s * RT, K)])
    pltpu.sync_copy(r0, agg_sp.at[pl.ds(s * RT + K, K)])
    pltpu.sync_copy(r0.at[pl.ds(0, RT - 2 * K)],
                    agg_sp.at[pl.ds(s * RT + 2 * K, RT - 2 * K)])
    plsc.subcore_barrier()

    for b in range(NB):
        unpack(b, b)
        pltpu.async_copy(y.at[gis[b]], rows[b], gs[b])

    def step(g, carry):
        i0 = NB * g
        for b in range(NB):
            pltpu.make_async_copy(y.at[pl.ds(0, K)], rows[b], gs[b]).wait()
            pltpu.async_copy(rows[b], agg_sp.at[sis[b]], ss[b], add=True)
        for b in range(NB):
            pltpu.make_async_copy(rows[b], agg_sp.at[pl.ds(0, K)],
                                  ss[b]).wait()
            unpack(i0 + NB + b, b)
            pltpu.async_copy(y.at[gis[b]], rows[b], gs[b])
        return carry

    lax.fori_loop(0, nch // NB - 1, step, 0)
    for b in range(NB):
        pltpu.make_async_copy(y.at[pl.ds(0, K)], rows[b], gs[b]).wait()
        pltpu.async_copy(rows[b], agg_sp.at[sis[b]], ss[b], add=True)
    for b in range(NB):
        pltpu.make_async_copy(rows[b], agg_sp.at[pl.ds(0, K)], ss[b]).wait()

    plsc.subcore_barrier()
    for k in range(2):
        pltpu.sync_copy(agg_sp.at[pl.ds(s * RT + k * K, K)], r0)
        pltpu.sync_copy(r0, out.at[c, pl.ds(s * RT + k * K, K)])
    pltpu.sync_copy(agg_sp.at[pl.ds(s * RT + 2 * K, RT - 2 * K)],
                    r0.at[pl.ds(0, RT - 2 * K)])
    pltpu.sync_copy(r0.at[pl.ds(0, RT - 2 * K)],
                    out.at[c, pl.ds(s * RT + 2 * K, RT - 2 * K)])


_scat = pl.kernel(
    _scat_body,
    out_type=jax.ShapeDtypeStruct((2, NR, D), jnp.float32),
    mesh=_mesh,
    scratch_types=(
        [pltpu.VMEM((SLAB + NB * K,), jnp.int32)]
        + [pltpu.VMEM((K,), jnp.int32) for _ in range(2 * NB)]
        + [pltpu.VMEM((K, D), jnp.float32) for _ in range(NB)]
        + [pltpu.SemaphoreType.DMA for _ in range(2 * NB)]
        + [pltpu.VMEM_SHARED((NR, D), jnp.float32)]
    ),
    compiler_params=pltpu.CompilerParams(needs_layout_passes=False),
)


def _inv_sqrt_deg(d):
    return jnp.where(d > 0, lax.rsqrt(jnp.maximum(d, 1.0)), 0.0)


def _mm1_body(x_ref, dout_ref, w_ref, o_ref):
    no = _inv_sqrt_deg(dout_ref[...])
    o_ref[...] = jnp.dot(x_ref[...] * no, w_ref[...],
                         preferred_element_type=jnp.float32,
                         precision=lax.Precision.HIGHEST)


def _mid_body(agg_ref, din_ref, dout_ref, b_ref, w_ref, o_ref):
    a = agg_ref[0]
    ni = _inv_sqrt_deg(din_ref[...])
    h1 = jnp.maximum(a * ni + b_ref[...], 0.0)
    no = _inv_sqrt_deg(dout_ref[...])
    o_ref[...] = jnp.dot(h1 * no, w_ref[...],
                         preferred_element_type=jnp.float32,
                         precision=lax.Precision.HIGHEST)


def _fin_body(agg_ref, din_ref, b_ref, o_ref):
    a = agg_ref[0]
    ni = _inv_sqrt_deg(din_ref[...])
    o_ref[...] = jnp.maximum(a * ni + b_ref[...], 0.0)


_col_spec = pl.BlockSpec((BR, 1), lambda i: (i, 0))
_row_spec = pl.BlockSpec((BR, D), lambda i: (i, 0))
_agg_spec = pl.BlockSpec((1, BR, D),
                         lambda i: (i // (NH // BR), i % (NH // BR), 0))
_w_spec = pl.BlockSpec((D, D), lambda i: (0, 0))
_b_spec = pl.BlockSpec((1, D), lambda i: (0, 0))
_out_sds = jax.ShapeDtypeStruct((N, D), jnp.float32)

_mm1 = pl.pallas_call(
    _mm1_body, grid=(GRID,),
    in_specs=[_row_spec, _col_spec, _w_spec],
    out_specs=_row_spec, out_shape=_out_sds)

_mid = pl.pallas_call(
    _mid_body, grid=(GRID,),
    in_specs=[_agg_spec, _col_spec, _col_spec, _b_spec, _w_spec],
    out_specs=_row_spec, out_shape=_out_sds)

_fin = pl.pallas_call(
    _fin_body, grid=(GRID,),
    in_specs=[_agg_spec, _col_spec, _b_spec],
    out_specs=_row_spec, out_shape=_out_sds)


def kernel(h, edge_index, W1, b1, W2, b2):
    src = edge_index[0].astype(jnp.int32)
    dst = edge_index[1].astype(jnp.int32)
    npad = EP - E
    pk = jnp.concatenate([
        jnp.left_shift(src, 14) | dst,
        jnp.full((npad,), N, jnp.int32),
    ])

    degs = _deg(pk)
    dout = degs[0].reshape(NP_, 1)
    din = degs[1].reshape(NP_, 1)

    b1r = b1.reshape(1, D)
    b2r = b2.reshape(1, D)

    y1 = _mm1(h, dout, W1)
    agg1 = _scat(y1, pk)
    y2 = _mid(agg1, din, dout, b1r, W2)
    agg2 = _scat(y2, pk)
    return _fin(agg2, din, b2r)
